# Initial kernel scaffold; baseline (speedup 1.0000x reference)
import jax
import jax.numpy as jnp
from jax.experimental import pallas as pl


def kernel(x, codebooks):
    raise NotImplementedError("write your pallas kernel here")


# speedup vs baseline: 1.0984x; 1.0984x over previous
import functools

import jax
import jax.numpy as jnp
from jax import lax
from jax.experimental import pallas as pl
from jax.experimental.pallas import tpu as pltpu
from jax.experimental.pallas import tpu_sc as plsc


def _argmin_body(KBLK, KC, K, x_ref, cb_ref, idx_ref, idxo_ref, loss_ref,
                 mval, midx):
    k = pl.program_id(2)
    cb = cb_ref[0]
    xb = x_ref[0, 0]
    cn = jnp.sum(cb * cb, axis=1, keepdims=True)
    xn = jnp.sum(xb * xb, axis=0, keepdims=True)
    mm = jnp.dot(cb, xb, preferred_element_type=jnp.float32)
    d2 = (xn - 2.0 * mm) + cn
    dist = jnp.sqrt(jnp.maximum(d2, 0.0))
    m = jnp.min(dist, axis=0, keepdims=True)
    kio = lax.broadcasted_iota(jnp.int32, dist.shape, 0) + k * KBLK
    li = jnp.min(jnp.where(dist == m, kio, K), axis=0, keepdims=True)

    @pl.when(k == 0)
    def _():
        mval[...] = m
        midx[...] = li

    @pl.when(k > 0)
    def _():
        better = m < mval[...]
        mval[...] = jnp.where(better, m, mval[...])
        midx[...] = jnp.where(better, li, midx[...])

    @pl.when(k == KC - 1)
    def _():
        i = pl.program_id(1)
        idx_ref[0, 0] = midx[...]
        idxo_ref[0, 0] = midx[...] + i * K
        md = mval[...]
        loss_ref[0, 0, 0, 0] = jnp.sum(md * md)


def _argmin_call(x4, codebooks, KBLK=1024, interpret=False):
    B, NCB, D, T = x4.shape
    _, K, _ = codebooks.shape
    KC = K // KBLK
    grid = (B, NCB, KC)
    body = functools.partial(_argmin_body, KBLK, KC, K)
    return pl.pallas_call(
        body,
        grid=grid,
        in_specs=[
            pl.BlockSpec((1, 1, D, T), lambda b, i, k: (b, i, 0, 0)),
            pl.BlockSpec((1, KBLK, D), lambda b, i, k: (i, k, 0)),
        ],
        out_specs=[
            pl.BlockSpec((1, 1, 1, T), lambda b, i, k: (b, i, 0, 0)),
            pl.BlockSpec((1, 1, 1, T), lambda b, i, k: (i, b, 0, 0)),
            pl.BlockSpec((1, 1, 1, 1), lambda b, i, k: (b, i, 0, 0),
                         memory_space=pltpu.SMEM),
        ],
        out_shape=[
            jax.ShapeDtypeStruct((B, NCB, 1, T), jnp.int32),
            jax.ShapeDtypeStruct((NCB, B, 1, T), jnp.int32),
            jax.ShapeDtypeStruct((B, NCB, 1, 1), jnp.float32),
        ],
        scratch_shapes=[
            pltpu.VMEM((1, T), jnp.float32),
            pltpu.VMEM((1, T), jnp.int32),
        ],
        interpret=interpret,
    )(x4, codebooks)


def _make_sc_gather(NROWS, D):
    info = plsc.get_sparse_core_info()
    NC, NS = info.num_cores, info.num_subcores
    NW = NC * NS
    rows_per_w = NROWS // NW
    CH = 512
    NCHUNK = rows_per_w // CH
    mesh = plsc.VectorSubcoreMesh(core_axis_name="c", subcore_axis_name="s")

    @functools.partial(
        pl.kernel, mesh=mesh,
        out_type=jax.ShapeDtypeStruct((NROWS, D), jnp.float32),
        scratch_types=[
            pltpu.VMEM((CH,), jnp.int32),
            pltpu.VMEM((CH, D), jnp.float32),
            pltpu.SemaphoreType.DMA,
        ],
    )
    def gather(table_hbm, idx_hbm, out_hbm, idx_v, rows_v, sem):
        wid = lax.axis_index("s") * NC + lax.axis_index("c")

        def body(c, carry):
            base = wid * rows_per_w + c * CH
            pltpu.sync_copy(idx_hbm.at[pl.ds(base, CH)], idx_v)
            pltpu.async_copy(table_hbm.at[idx_v], rows_v, sem).wait()
            pltpu.sync_copy(rows_v, out_hbm.at[pl.ds(base, CH)])
            return carry

        lax.fori_loop(0, NCHUNK, body, 0)

    return gather


def kernel(x, codebooks):
    B, C, T = x.shape
    NCB, K, D = codebooks.shape
    x4 = x.reshape(B, NCB, D, T)

    idx4, idxo, loss_parts = _argmin_call(x4, codebooks)

    NROWS = NCB * B * T
    table = codebooks.reshape(NCB * K, D)
    gather = _make_sc_gather(NROWS, D)
    q = gather(table, idxo.reshape(NROWS))

    quantized = (q.reshape(NCB, B, T, D)
                  .transpose(1, 0, 3, 2)
                  .reshape(B, C, T))
    indices = idx4.reshape(B, NCB, T)
    loss = 0.25 * jnp.sum(loss_parts) / (B * T * D)
    return quantized, indices, loss


# speedup vs baseline: 1.4638x; 1.3327x over previous
import functools

import jax
import jax.numpy as jnp
from jax import lax
from jax.experimental import pallas as pl
from jax.experimental.pallas import tpu as pltpu
from jax.experimental.pallas import tpu_sc as plsc


def _succ(x):
    b = lax.bitcast_convert_type(x, jnp.int32)
    return lax.bitcast_convert_type(b + 1, jnp.float32)


def _argmin_body(KBLK, KC, K, x_ref, cb_ref, idx_ref, idxo_ref, loss_ref,
                 xn_row, rs, ridx, gmin):
    k = pl.program_id(2)
    cb = cb_ref[0]
    xb = x_ref[0, 0]
    cn = jnp.sum(cb * cb, axis=1, keepdims=True)

    @pl.when(k == 0)
    def _():
        xn_row[...] = jnp.sum(xb * xb, axis=0, keepdims=True)

    xn = xn_row[...]
    mm = jnp.dot(cb, xb, preferred_element_type=jnp.float32)
    d2 = (xn - 2.0 * mm) + cn
    bm = jnp.min(d2, axis=0, keepdims=True)

    bmc = jnp.maximum(bm, 0.0)
    s = jnp.sqrt(bmc)
    u = bmc
    x = bmc
    for _ in range(5):
        x = _succ(x)
        u = jnp.where(jnp.sqrt(x) == s, x, u)

    kio = lax.broadcasted_iota(jnp.int32, d2.shape, 0) + k * KBLK
    li = jnp.min(jnp.where(d2 <= u, kio, 2 * K),
                 axis=0, keepdims=True)

    @pl.when(k == 0)
    def _():
        rs[...] = s
        ridx[...] = li
        gmin[...] = bm

    @pl.when(k > 0)
    def _():
        rs_o = rs[...]
        better = s < rs_o
        equal = s == rs_o
        rs[...] = jnp.where(better, s, rs_o)
        ridx[...] = jnp.where(
            better, li,
            jnp.where(equal, jnp.minimum(ridx[...], li), ridx[...]))
        gmin[...] = jnp.minimum(gmin[...], bm)

    @pl.when(k == KC - 1)
    def _():
        i = pl.program_id(1)
        idx_i = ridx[...]
        idx_ref[0, 0] = idx_i
        idxo_ref[0, 0] = idx_i + i * K
        loss_ref[0, 0, 0, 0] = jnp.sum(gmin[...])


def _argmin_call(x4, codebooks, KBLK=1024, interpret=False):
    B, NCB, D, T = x4.shape
    _, K, _ = codebooks.shape
    KC = K // KBLK
    grid = (B, NCB, KC)
    body = functools.partial(_argmin_body, KBLK, KC, K)
    return pl.pallas_call(
        body,
        grid=grid,
        in_specs=[
            pl.BlockSpec((1, 1, D, T), lambda b, i, k: (b, i, 0, 0)),
            pl.BlockSpec((1, KBLK, D), lambda b, i, k: (i, k, 0)),
        ],
        out_specs=[
            pl.BlockSpec((1, 1, 1, T), lambda b, i, k: (b, i, 0, 0)),
            pl.BlockSpec((1, 1, 1, T), lambda b, i, k: (i, b, 0, 0)),
            pl.BlockSpec((1, 1, 1, 1), lambda b, i, k: (b, i, 0, 0),
                         memory_space=pltpu.SMEM),
        ],
        out_shape=[
            jax.ShapeDtypeStruct((B, NCB, 1, T), jnp.int32),
            jax.ShapeDtypeStruct((NCB, B, 1, T), jnp.int32),
            jax.ShapeDtypeStruct((B, NCB, 1, 1), jnp.float32),
        ],
        scratch_shapes=[
            pltpu.VMEM((1, T), jnp.float32),
            pltpu.VMEM((1, T), jnp.float32),
            pltpu.VMEM((1, T), jnp.int32),
            pltpu.VMEM((1, T), jnp.float32),
        ],
        interpret=interpret,
    )(x4, codebooks)


def _make_sc_gather(NROWS, D):
    info = plsc.get_sparse_core_info()
    NC, NS = info.num_cores, info.num_subcores
    NW = NC * NS
    rows_per_w = NROWS // NW
    CH = 512
    NCHUNK = rows_per_w // CH
    mesh = plsc.VectorSubcoreMesh(core_axis_name="c", subcore_axis_name="s")

    @functools.partial(
        pl.kernel, mesh=mesh,
        out_type=jax.ShapeDtypeStruct((NROWS, D), jnp.float32),
        scratch_types=[
            pltpu.VMEM((CH,), jnp.int32),
            pltpu.VMEM((CH, D), jnp.float32),
            pltpu.SemaphoreType.DMA,
        ],
    )
    def gather(table_hbm, idx_hbm, out_hbm, idx_v, rows_v, sem):
        wid = lax.axis_index("s") * NC + lax.axis_index("c")

        def body(c, carry):
            base = wid * rows_per_w + c * CH
            pltpu.sync_copy(idx_hbm.at[pl.ds(base, CH)], idx_v)
            pltpu.async_copy(table_hbm.at[idx_v], rows_v, sem).wait()
            pltpu.sync_copy(rows_v, out_hbm.at[pl.ds(base, CH)])
            return carry

        lax.fori_loop(0, NCHUNK, body, 0)

    return gather


def kernel(x, codebooks):
    B, C, T = x.shape
    NCB, K, D = codebooks.shape
    x4 = x.reshape(B, NCB, D, T)

    idx4, idxo, loss_parts = _argmin_call(x4, codebooks)

    NROWS = NCB * B * T
    table = codebooks.reshape(NCB * K, D)
    gather = _make_sc_gather(NROWS, D)
    q = gather(table, idxo.reshape(NROWS))

    quantized = (q.reshape(NCB, B, T, D)
                  .transpose(1, 0, 3, 2)
                  .reshape(B, C, T))
    indices = idx4.reshape(B, NCB, T)
    loss = 0.25 * jnp.sum(loss_parts) / (B * T * D)
    return quantized, indices, loss


# speedup vs baseline: 2.1361x; 1.4593x over previous
import functools

import jax
import jax.numpy as jnp
from jax import lax
from jax.experimental import pallas as pl
from jax.experimental.pallas import tpu as pltpu
from jax.experimental.pallas import tpu_sc as plsc


def _succ(x):
    b = lax.bitcast_convert_type(x, jnp.int32)
    return lax.bitcast_convert_type(b + 1, jnp.float32)


def _argmin_body(KBLK, KC, K, x_ref, cb_ref, idx_ref, idxo_ref, loss_ref,
                 xn_row, rs, ridx, gmin):
    k = pl.program_id(2)
    cb = cb_ref[0]
    xb = x_ref[0, 0]
    cn = jnp.sum(cb * cb, axis=1, keepdims=True)

    @pl.when(k == 0)
    def _():
        xn_row[...] = jnp.sum(xb * xb, axis=0, keepdims=True)

    xn = xn_row[...]
    mm2 = jnp.dot(cb, xb + xb, preferred_element_type=jnp.float32)
    d2 = (xn - mm2) + cn
    bm = jnp.min(d2, axis=0, keepdims=True)

    bmc = jnp.maximum(bm, 0.0)
    s = jnp.sqrt(bmc)
    u = bmc
    x = bmc
    for _ in range(5):
        x = _succ(x)
        u = jnp.where(jnp.sqrt(x) == s, x, u)

    kio = lax.broadcasted_iota(jnp.int32, d2.shape, 0)
    li = jnp.min(jnp.where(d2 <= u, kio, 2 * K),
                 axis=0, keepdims=True) + k * KBLK

    @pl.when(k == 0)
    def _():
        rs[...] = s
        ridx[...] = li
        gmin[...] = bm

    @pl.when(k > 0)
    def _():
        rs_o = rs[...]
        better = s < rs_o
        equal = s == rs_o
        rs[...] = jnp.where(better, s, rs_o)
        ridx[...] = jnp.where(
            better, li,
            jnp.where(equal, jnp.minimum(ridx[...], li), ridx[...]))
        gmin[...] = jnp.minimum(gmin[...], bm)

    @pl.when(k == KC - 1)
    def _():
        i = pl.program_id(1)
        idx_i = ridx[...]
        idx_ref[0, 0] = idx_i
        idxo_ref[0, 0] = idx_i + i * K
        loss_ref[0, 0, 0, 0] = jnp.sum(gmin[...])


def _argmin_call(x4, codebooks, KBLK=8192, interpret=False):
    B, NCB, D, T = x4.shape
    _, K, _ = codebooks.shape
    KC = K // KBLK
    grid = (B, NCB, KC)
    body = functools.partial(_argmin_body, KBLK, KC, K)
    return pl.pallas_call(
        body,
        grid=grid,
        in_specs=[
            pl.BlockSpec((1, 1, D, T), lambda b, i, k: (b, i, 0, 0)),
            pl.BlockSpec((1, KBLK, D), lambda b, i, k: (i, k, 0)),
        ],
        out_specs=[
            pl.BlockSpec((1, 1, 1, T), lambda b, i, k: (b, i, 0, 0)),
            pl.BlockSpec((1, 1, 1, T), lambda b, i, k: (i, b, 0, 0)),
            pl.BlockSpec((1, 1, 1, 1), lambda b, i, k: (b, i, 0, 0),
                         memory_space=pltpu.SMEM),
        ],
        out_shape=[
            jax.ShapeDtypeStruct((B, NCB, 1, T), jnp.int32),
            jax.ShapeDtypeStruct((NCB, B, 1, T), jnp.int32),
            jax.ShapeDtypeStruct((B, NCB, 1, 1), jnp.float32),
        ],
        scratch_shapes=[
            pltpu.VMEM((1, T), jnp.float32),
            pltpu.VMEM((1, T), jnp.float32),
            pltpu.VMEM((1, T), jnp.int32),
            pltpu.VMEM((1, T), jnp.float32),
        ],
        interpret=interpret,
    )(x4, codebooks)


def _make_sc_gather(NROWS, D):
    info = plsc.get_sparse_core_info()
    NC, NS = info.num_cores, info.num_subcores
    NW = NC * NS
    rows_per_w = NROWS // NW
    CH = 512
    NCHUNK = rows_per_w // CH
    mesh = plsc.VectorSubcoreMesh(core_axis_name="c", subcore_axis_name="s")

    @functools.partial(
        pl.kernel, mesh=mesh,
        out_type=jax.ShapeDtypeStruct((NROWS, D), jnp.float32),
        scratch_types=[
            pltpu.VMEM((CH,), jnp.int32),
            pltpu.VMEM((CH, D), jnp.float32),
            pltpu.SemaphoreType.DMA,
        ],
    )
    def gather(table_hbm, idx_hbm, out_hbm, idx_v, rows_v, sem):
        wid = lax.axis_index("s") * NC + lax.axis_index("c")

        def body(c, carry):
            base = wid * rows_per_w + c * CH
            pltpu.sync_copy(idx_hbm.at[pl.ds(base, CH)], idx_v)
            pltpu.async_copy(table_hbm.at[idx_v], rows_v, sem).wait()
            pltpu.sync_copy(rows_v, out_hbm.at[pl.ds(base, CH)])
            return carry

        lax.fori_loop(0, NCHUNK, body, 0)

    return gather


def kernel(x, codebooks):
    B, C, T = x.shape
    NCB, K, D = codebooks.shape
    x4 = x.reshape(B, NCB, D, T)

    idx4, idxo, loss_parts = _argmin_call(x4, codebooks)

    NROWS = NCB * B * T
    table = codebooks.reshape(NCB * K, D)
    gather = _make_sc_gather(NROWS, D)
    q = gather(table, idxo.reshape(NROWS))

    quantized = (q.reshape(NCB, B, T, D)
                  .transpose(1, 0, 3, 2)
                  .reshape(B, C, T))
    indices = idx4.reshape(B, NCB, T)
    loss = 0.25 * jnp.sum(loss_parts) / (B * T * D)
    return quantized, indices, loss


# speedup vs baseline: 2.1971x; 1.0285x over previous
import functools

import jax
import jax.numpy as jnp
from jax import lax
from jax.experimental import pallas as pl
from jax.experimental.pallas import tpu as pltpu
from jax.experimental.pallas import tpu_sc as plsc


def _succ(x):
    b = lax.bitcast_convert_type(x, jnp.int32)
    return lax.bitcast_convert_type(b + 1, jnp.float32)


def _argmin_body(K, x_ref, cb_ref, idx_ref, idxo_ref, loss_ref, li_row):
    cb = cb_ref[0]
    xb = x_ref[0, 0]
    cn = jnp.sum(cb * cb, axis=1, keepdims=True)
    xn = jnp.sum(xb * xb, axis=0, keepdims=True)
    mm2 = jnp.dot(cb, xb + xb, preferred_element_type=jnp.float32)
    d2 = (xn - mm2) + cn
    bm = jnp.min(d2, axis=0, keepdims=True)

    bmc = jnp.maximum(bm, 0.0)
    s = jnp.sqrt(bmc)
    u = bmc
    x = bmc
    for _ in range(5):
        x = _succ(x)
        u = jnp.where(jnp.sqrt(x) == s, x, u)

    maskf = jnp.where(d2 <= u, 1.0, 0.0)
    rr = lax.broadcasted_iota(jnp.int32, (8, K), 0)
    cc = lax.broadcasted_iota(jnp.int32, (8, K), 1)
    w = jnp.where(rr == 0, cc, 1).astype(jnp.float32)
    ext = jnp.dot(w[:2], maskf, preferred_element_type=jnp.float32)
    li_main = ext[0:1].astype(jnp.int32)
    cnt = ext[1:2]

    @pl.when(jnp.max(cnt) <= 1.5)
    def _():
        li_row[...] = li_main

    @pl.when(jnp.max(cnt) > 1.5)
    def _():
        kio = lax.broadcasted_iota(jnp.int32, maskf.shape, 0)
        li_row[...] = jnp.min(jnp.where(maskf > 0.5, kio, 2 * K),
                              axis=0, keepdims=True)

    i = pl.program_id(1)
    li = li_row[...]
    idx_ref[0, 0] = li
    idxo_ref[0, 0] = li + i * K
    loss_ref[0, 0, 0] = jnp.sum(bm)


def _argmin_call(x4, codebooks, TBLK=512, interpret=False):
    B, NCB, D, T = x4.shape
    _, K, _ = codebooks.shape
    NT = T // TBLK
    grid = (B, NCB, NT)
    body = functools.partial(_argmin_body, K)
    return pl.pallas_call(
        body,
        grid=grid,
        in_specs=[
            pl.BlockSpec((1, 1, D, TBLK), lambda b, i, t: (b, i, 0, t)),
            pl.BlockSpec((1, K, D), lambda b, i, t: (i, 0, 0)),
        ],
        out_specs=[
            pl.BlockSpec((1, 1, 1, TBLK), lambda b, i, t: (b, i, 0, t)),
            pl.BlockSpec((1, 1, 1, TBLK), lambda b, i, t: (i, b, 0, t)),
            pl.BlockSpec((1, 1, 1), lambda b, i, t: ((b * NCB + i) * NT + t,
                                                     0, 0),
                         memory_space=pltpu.SMEM),
        ],
        out_shape=[
            jax.ShapeDtypeStruct((B, NCB, 1, T), jnp.int32),
            jax.ShapeDtypeStruct((NCB, B, 1, T), jnp.int32),
            jax.ShapeDtypeStruct((B * NCB * NT, 1, 1), jnp.float32),
        ],
        scratch_shapes=[
            pltpu.VMEM((1, TBLK), jnp.int32),
        ],
        interpret=interpret,
    )(x4, codebooks)


def _make_sc_gather(NROWS, D):
    info = plsc.get_sparse_core_info()
    NC, NS = info.num_cores, info.num_subcores
    NW = NC * NS
    rows_per_w = NROWS // NW
    CH = 512
    NCHUNK = rows_per_w // CH
    mesh = plsc.VectorSubcoreMesh(core_axis_name="c", subcore_axis_name="s")

    @functools.partial(
        pl.kernel, mesh=mesh,
        out_type=jax.ShapeDtypeStruct((NROWS, D), jnp.float32),
        scratch_types=[
            pltpu.VMEM((CH,), jnp.int32),
            pltpu.VMEM((CH, D), jnp.float32),
            pltpu.SemaphoreType.DMA,
        ],
    )
    def gather(table_hbm, idx_hbm, out_hbm, idx_v, rows_v, sem):
        wid = lax.axis_index("s") * NC + lax.axis_index("c")

        def body(c, carry):
            base = wid * rows_per_w + c * CH
            pltpu.sync_copy(idx_hbm.at[pl.ds(base, CH)], idx_v)
            pltpu.async_copy(table_hbm.at[idx_v], rows_v, sem).wait()
            pltpu.sync_copy(rows_v, out_hbm.at[pl.ds(base, CH)])
            return carry

        lax.fori_loop(0, NCHUNK, body, 0)

    return gather


def kernel(x, codebooks):
    B, C, T = x.shape
    NCB, K, D = codebooks.shape
    x4 = x.reshape(B, NCB, D, T)

    idx4, idxo, loss_parts = _argmin_call(x4, codebooks)

    NROWS = NCB * B * T
    table = codebooks.reshape(NCB * K, D)
    gather = _make_sc_gather(NROWS, D)
    q = gather(table, idxo.reshape(NROWS))

    quantized = (q.reshape(NCB, B, T, D)
                  .transpose(1, 0, 3, 2)
                  .reshape(B, C, T))
    indices = idx4.reshape(B, NCB, T)
    loss = 0.25 * jnp.sum(loss_parts) / (B * T * D)
    return quantized, indices, loss
